# baseline (speedup 1.0000x reference)
import functools

import jax
import jax.numpy as jnp
from jax import lax
from jax.experimental import pallas as pl
from jax.experimental.pallas import tpu as pltpu
from jax.experimental.pallas import tpu_sc as plsc

NC = 2
NS = 16
NW = NC * NS
BATCH = 4096
HIST = 50
D = 64
V = 89
IPT = BATCH // NW
NGRP = 5

_mesh = plsc.VectorSubcoreMesh(
    core_axis_name="c", subcore_axis_name="s", num_cores=NC, num_subcores=NS
)


@functools.partial(
    pl.kernel,
    out_type=jax.ShapeDtypeStruct((HIST, D // 8, NW, 8, IPT), jnp.float32),
    mesh=_mesh,
    scratch_types=(
        [pltpu.VMEM((HIST, IPT), jnp.int32)]
        + [pltpu.VMEM((NGRP, IPT, D), jnp.float32)]
        + [pltpu.VMEM((NGRP, D // 8, 8, IPT + 1), jnp.float32)]
        + [pltpu.SemaphoreType.DMA] * (1 + 2 * NGRP)
    ),
    compiler_params=pltpu.CompilerParams(
        use_tc_tiling_on_sc=False, needs_layout_passes=False
    ),
)
def _emb_lookup(trep_hbm, idx_hbm, out_hbm, idx_v, gbuf, tbuf, isem, *sems):
    gsems = sems[:NGRP]
    ssems = sems[NGRP:]
    wid = lax.axis_index("s") * NC + lax.axis_index("c")

    pltpu.async_copy(idx_hbm.at[wid], idx_v, isem).wait()

    tab = trep_hbm.at[wid]

    def gather_desc(h, p):
        return pltpu.make_async_copy(
            tab.at[idx_v.at[h]], gbuf.at[p], gsems[p]
        )

    def scatter_desc(h, p):
        return pltpu.make_async_copy(
            tbuf.at[p].at[:, :, pl.ds(0, IPT)], out_hbm.at[h, :, wid], ssems[p]
        )

    iota = lax.iota(jnp.int32, 16)
    dt_s = [(dd * 16 + iota) >> 3 for dd in range(D // 16)]
    di_s = [(dd * 16 + iota) & 7 for dd in range(D // 16)]

    def transpose_chunk(p):
        gsrc = gbuf.at[p]
        tdst = tbuf.at[p]

        def item8(b8, c):
            b0 = b8 * 8
            for u in range(8):
                b = b0 + u
                bvec = jnp.broadcast_to(b, (16,))
                for dd in range(D // 16):
                    v = gsrc[b, pl.ds(dd * 16, 16)]
                    plsc.store_scatter(tdst, [dt_s[dd], di_s[dd], bvec], v)
            return c

        lax.fori_loop(0, IPT // 8, item8, 0)

    def phase(h, p, prefetch=True, reuse_wait=True):
        gather_desc(h, p).wait()
        transpose_chunk(p)
        scatter_desc(h, p).start()
        f = h + 2
        if prefetch:
            pf = (p + 2) % NGRP
            if reuse_wait:
                scatter_desc(f - NGRP, pf).wait()
            gather_desc(f, pf).start()

    gather_desc(0, 0).start()
    gather_desc(1, 1).start()
    phase(0, 0, reuse_wait=False)
    phase(1, 1, reuse_wait=False)
    phase(2, 2, reuse_wait=False)
    phase(3, 3)
    phase(4, 4)

    def body(i, c):
        h = 5 * i
        for q in range(NGRP):
            phase(h + q, q)
        return c

    lax.fori_loop(1, 9, body, 0)

    phase(45, 0)
    phase(46, 1)
    phase(47, 2)
    phase(48, 3, prefetch=False)
    phase(49, 4, prefetch=False)
    for q in range(NGRP):
        scatter_desc(45 + q, q).wait()


def kernel(x, table):
    if x.ndim > 1 and x.shape[-1] == 1:
        x = x[..., 0]
    trep = jnp.tile(table.astype(jnp.float32)[None], (NW, 1, 1))
    idx3 = x.astype(jnp.int32).reshape(NW, IPT, HIST).transpose(0, 2, 1)
    o5 = _emb_lookup(trep, idx3)
    return o5.transpose(2, 4, 0, 1, 3).reshape(BATCH, HIST, D)


# speedup vs baseline: 1.0433x; 1.0433x over previous
import functools

import jax
import jax.numpy as jnp
from jax import lax
from jax.experimental import pallas as pl
from jax.experimental.pallas import tpu as pltpu
from jax.experimental.pallas import tpu_sc as plsc

NC = 2
NS = 16
NW = NC * NS
BATCH = 4096
HIST = 50
D = 64
V = 89
IPT = BATCH // NW
NGRP = 3

_mesh = plsc.VectorSubcoreMesh(
    core_axis_name="c", subcore_axis_name="s", num_cores=NC, num_subcores=NS
)


@functools.partial(
    pl.kernel,
    out_type=jax.ShapeDtypeStruct((HIST, D // 8, NW, 8, IPT), jnp.float32),
    mesh=_mesh,
    scratch_types=(
        [pltpu.VMEM((HIST, IPT), jnp.int32)]
        + [pltpu.VMEM((NGRP, IPT, D), jnp.float32)]
        + [pltpu.VMEM((NGRP, D // 8, 8, IPT + 1), jnp.float32)]
        + [pltpu.SemaphoreType.DMA] * (1 + 2 * NGRP)
    ),
    compiler_params=pltpu.CompilerParams(
        use_tc_tiling_on_sc=False, needs_layout_passes=False
    ),
)
def _emb_lookup(trep_hbm, idx_hbm, out_hbm, idx_v, gbuf, tbuf, isem, *sems):
    gsems = sems[:NGRP]
    ssems = sems[NGRP:]
    wid = lax.axis_index("s") * NC + lax.axis_index("c")

    pltpu.async_copy(idx_hbm.at[wid], idx_v, isem).wait()

    tab = trep_hbm.at[wid]

    def gather_desc(h, p):
        return pltpu.make_async_copy(
            tab.at[idx_v.at[h]], gbuf.at[p], gsems[p]
        )

    def scatter_desc(h, p):
        return pltpu.make_async_copy(
            tbuf.at[p].at[:, :, pl.ds(0, IPT)], out_hbm.at[h, :, wid], ssems[p]
        )

    iota = lax.iota(jnp.int32, 16)
    dt_s = [(dd * 16 + iota) >> 3 for dd in range(D // 16)]
    di_s = [(dd * 16 + iota) & 7 for dd in range(D // 16)]

    def transpose_chunk(p):
        gsrc = gbuf.at[p]
        tdst = tbuf.at[p]

        def item8(b8, c):
            b0 = b8 * 8
            for u in range(8):
                b = b0 + u
                bvec = jnp.broadcast_to(b, (16,))
                for dd in range(D // 16):
                    v = gsrc[b, pl.ds(dd * 16, 16)]
                    plsc.store_scatter(tdst, [dt_s[dd], di_s[dd], bvec], v)
            return c

        lax.fori_loop(0, IPT // 8, item8, 0)

    def phase(h, p, prefetch=True, reuse_wait=True):
        gather_desc(h, p).wait()
        transpose_chunk(p)
        scatter_desc(h, p).start()
        f = h + 2
        if prefetch:
            pf = (p + 2) % NGRP
            if reuse_wait:
                scatter_desc(f - NGRP, pf).wait()
            gather_desc(f, pf).start()

    gather_desc(0, 0).start()
    gather_desc(1, 1).start()
    phase(0, 0, reuse_wait=False)
    phase(1, 1)
    phase(2, 2)

    def body(i, c):
        h = 3 * i
        for q in range(NGRP):
            phase(h + q, q)
        return c

    lax.fori_loop(1, 16, body, 0)

    phase(48, 0, prefetch=False)
    phase(49, 1, prefetch=False)

    scatter_desc(47, 2).wait()
    scatter_desc(48, 0).wait()
    scatter_desc(49, 1).wait()


def kernel(x, table):
    if x.ndim > 1 and x.shape[-1] == 1:
        x = x[..., 0]
    trep = jnp.tile(table.astype(jnp.float32)[None], (NW, 1, 1))
    idx3 = x.astype(jnp.int32).reshape(NW, IPT, HIST).transpose(0, 2, 1)
    o5 = _emb_lookup(trep, idx3)
    return o5.transpose(2, 4, 0, 1, 3).reshape(BATCH, HIST, D)
